# baseline (speedup 1.0000x reference)
import functools

import jax
import jax.numpy as jnp
from jax import lax
from jax.experimental import pallas as pl
from jax.experimental.pallas import tpu as pltpu
from jax.experimental.pallas import tpu_sc as plsc

NC = 2
NS = 16
SUB = 3
CHUNK = SUB * 128


def _sc_mesh():
    return plsc.VectorSubcoreMesh(core_axis_name="c", subcore_axis_name="s")


_SC_PARAMS = pltpu.CompilerParams(use_tc_tiling_on_sc=False)


@functools.partial(jax.jit, static_argnames=("n", "nacc", "iters"))
def _sc_degree(idx2, ones_h, zeros_h, *, n, nacc, iters):
    zr = nacc // NS
    rows_per_tile = iters * SUB

    def body(idx_hbm, ones_hbm, zeros_hbm, out_hbm, idxv, ones_v, acc, sem):
        c = lax.axis_index("c")
        s = lax.axis_index("s")
        pltpu.sync_copy(ones_hbm, ones_v)
        pltpu.sync_copy(zeros_hbm, acc.at[pl.ds(s * zr, zr)])
        plsc.subcore_barrier()

        def idx_src(i):
            return idx_hbm.at[c, pl.ds(s * rows_per_tile + i * SUB, SUB)]

        pltpu.sync_copy(idx_src(0), idxv.at[0])

        def pair(p, carry):
            for b in (0, 1):
                i = 2 * p + b

                @pl.when(i + 1 < iters)
                def _():
                    pltpu.async_copy(idx_src(i + 1), idxv.at[1 - b], sem)

                for j in range(SUB):
                    pltpu.sync_copy(ones_v, acc.at[idxv.at[b, j]], add=True)

                @pl.when(i + 1 < iters)
                def _():
                    pltpu.make_async_copy(idx_src(i + 1), idxv.at[1 - b],
                                          sem).wait()
            return carry

        lax.fori_loop(0, iters // 2, pair, 0)
        plsc.subcore_barrier()
        r0 = s * zr
        last = n - (NS - 1) * zr

        @pl.when(s < NS - 1)
        def _():
            pltpu.sync_copy(acc.at[pl.ds(r0, zr)], out_hbm.at[c, pl.ds(r0, zr)])

        @pl.when(s == NS - 1)
        def _():
            pltpu.sync_copy(acc.at[pl.ds(r0, last)],
                            out_hbm.at[c, pl.ds(r0, last)])

    fn = pl.kernel(
        body,
        out_type=jax.ShapeDtypeStruct((2, n, 16), jnp.float32),
        mesh=_sc_mesh(),
        scratch_types=[
            pltpu.VMEM((2, SUB, 128), jnp.int32),
            pltpu.VMEM((128, 16), jnp.float32),
            pltpu.VMEM_SHARED((nacc, 16), jnp.float32),
            pltpu.SemaphoreType.DMA,
        ],
        compiler_params=_SC_PARAMS,
    )
    return fn(idx2, ones_h, zeros_h)


@functools.partial(jax.jit, static_argnames=("n", "nacc", "iters"))
def _sc_gather_scatter(h2, src2, dst2, zeros_h, *, n, nacc, iters):
    zr = nacc // NS
    rows_per_tile = iters * SUB

    def body(h_hbm, src_hbm, dst_hbm, zeros_hbm, out_hbm,
             sidx, didx, rows_v, acc, gsem):
        c = lax.axis_index("c")
        s = lax.axis_index("s")
        pltpu.sync_copy(zeros_hbm, acc.at[pl.ds(s * zr, zr)])
        plsc.subcore_barrier()

        def load_idx(i, q, b):
            base = s * rows_per_tile + i * SUB
            pltpu.sync_copy(src_hbm.at[c, pl.ds(base, SUB)], sidx.at[q, b])
            pltpu.sync_copy(dst_hbm.at[pl.ds(base, SUB)], didx.at[q, b])

        def fire_gathers(q, b):
            for j in range(SUB):
                pltpu.async_copy(h_hbm.at[sidx.at[q, b, j]],
                                 rows_v.at[b, pl.ds(j * 128, 128)], gsem)

        def drain_gathers(b):
            pltpu.make_async_copy(h_hbm.at[pl.ds(0, CHUNK)], rows_v.at[b],
                                  gsem).wait()

        load_idx(0, 0, 0)
        load_idx(1, 0, 1)
        fire_gathers(0, 0)

        def quad(p2, carry):
            for pp in (0, 1):
                for b in (0, 1):
                    i = 4 * p2 + 2 * pp + b
                    q = pp
                    nq, nb = (q, 1 - b) if b == 0 else (1 - q, 1 - b)

                    @pl.when(i + 1 < iters)
                    def _():
                        fire_gathers(nq, nb)

                    @pl.when(i + 2 < iters)
                    def _():
                        load_idx(i + 2, 1 - q, b)

                    drain_gathers(b)
                    for j in range(SUB):
                        pltpu.sync_copy(rows_v.at[b, pl.ds(j * 128, 128)],
                                        acc.at[didx.at[q, b, j]], add=True)
            return carry

        lax.fori_loop(0, iters // 4, quad, 0)
        plsc.subcore_barrier()
        r0 = s * zr
        last = n - (NS - 1) * zr

        @pl.when(s < NS - 1)
        def _():
            pltpu.sync_copy(acc.at[pl.ds(r0, zr)], out_hbm.at[c, pl.ds(r0, zr)])

        @pl.when(s == NS - 1)
        def _():
            pltpu.sync_copy(acc.at[pl.ds(r0, last)],
                            out_hbm.at[c, pl.ds(r0, last)])

    fn = pl.kernel(
        body,
        out_type=jax.ShapeDtypeStruct((2, n, 32), jnp.float32),
        mesh=_sc_mesh(),
        scratch_types=[
            pltpu.VMEM((2, 2, SUB, 128), jnp.int32),
            pltpu.VMEM((2, 2, SUB, 128), jnp.int32),
            pltpu.VMEM((2, CHUNK, 32), jnp.float32),
            pltpu.VMEM_SHARED((nacc, 32), jnp.float32),
            pltpu.SemaphoreType.DMA,
        ],
        compiler_params=_SC_PARAMS,
    )
    return fn(h2, src2, dst2, zeros_h)


_RP = 256


def _normp(deg_blk):
    return lax.rsqrt(jnp.maximum(deg_blk, 1.0))


def _tc_pre_body(x_ref, w_ref, b_ref, d_ref, o_ref):
    xb = x_ref[...]
    ns = _normp(d_ref[...])
    for c in (0, 1):
        h = jnp.tanh(jnp.dot(xb, w_ref[c], preferred_element_type=jnp.float32)
                     + b_ref[c])
        o_ref[c] = h * ns


def _tc_layer_body(a_ref, w_ref, b_ref, dd_ref, ds_ref, o_ref):
    nd = _normp(dd_ref[...])
    ns = _normp(ds_ref[...])
    a0 = a_ref[0] * nd
    a1 = a_ref[1] * nd
    for c in (0, 1):
        h = jnp.tanh(jnp.dot(a0, w_ref[0, c], preferred_element_type=jnp.float32)
                     + jnp.dot(a1, w_ref[1, c], preferred_element_type=jnp.float32)
                     + b_ref[c])
        o_ref[c] = h * ns


def _tc_final_body(a_ref, w2_ref, b2_ref, wd_ref, bd_ref, wc_ref, bc_ref,
                   dd_ref, o_ref):
    nd = _normp(dd_ref[...])
    a0 = a_ref[0] * nd
    a1 = a_ref[1] * nd
    g = [jnp.dot(a0, w2_ref[0, c], preferred_element_type=jnp.float32)
         + jnp.dot(a1, w2_ref[1, c], preferred_element_type=jnp.float32)
         + b2_ref[c] for c in (0, 1)]
    t = [jnp.tanh(jnp.dot(g[0], wd_ref[0, c], preferred_element_type=jnp.float32)
                  + jnp.dot(g[1], wd_ref[1, c], preferred_element_type=jnp.float32)
                  + bd_ref[c]) for c in (0, 1)]
    o_ref[...] = (jnp.dot(t[0], wc_ref[0], preferred_element_type=jnp.float32)
                  + jnp.dot(t[1], wc_ref[1], preferred_element_type=jnp.float32)
                  + bc_ref[...])


def _rp_spec(cols):
    return pl.BlockSpec((_RP, cols), lambda i: (i, 0))


def _fullb(*shape):
    nd = len(shape)
    return pl.BlockSpec(shape, lambda i, _nd=nd: (0,) * _nd)


def _packed_spec():
    return pl.BlockSpec((2, _RP, 128), lambda i: (0, i, 0))


def _tc_pre(x4, wbd, b2x, degp, n):
    return pl.pallas_call(
        _tc_pre_body,
        grid=(-(-(n // 4) // _RP),),
        in_specs=[_rp_spec(x4.shape[1]), _fullb(*wbd.shape),
                  _fullb(2, 1, 128), _rp_spec(128)],
        out_specs=_packed_spec(),
        out_shape=jax.ShapeDtypeStruct((2, n // 4, 128), jnp.float32),
    )(x4, wbd, b2x, degp)


def _tc_layer(a, wbd, b2x, degdp, degsp, n):
    return pl.pallas_call(
        _tc_layer_body,
        grid=(-(-(n // 4) // _RP),),
        in_specs=[_packed_spec(), _fullb(*wbd.shape), _fullb(2, 1, 128),
                  _rp_spec(128), _rp_spec(128)],
        out_specs=_packed_spec(),
        out_shape=jax.ShapeDtypeStruct((2, n // 4, 128), jnp.float32),
    )(a, wbd, b2x, degdp, degsp)


def _tc_final(a, w2bd, b2x, wdbd, bdx, wcbd, bcx, degdp, n):
    cols = wcbd.shape[2]
    return pl.pallas_call(
        _tc_final_body,
        grid=(-(-(n // 4) // _RP),),
        in_specs=[_packed_spec(), _fullb(*w2bd.shape), _fullb(2, 1, 128),
                  _fullb(*wdbd.shape), _fullb(2, 1, 128),
                  _fullb(*wcbd.shape), _fullb(1, cols), _rp_spec(128)],
        out_specs=pl.BlockSpec((_RP, cols), lambda i: (i, 0)),
        out_shape=jax.ShapeDtypeStruct((n // 4, cols), jnp.float32),
    )(a, w2bd, b2x, wdbd, bdx, wcbd, bcx, degdp)


def kernel(x, edge_index, W_in, b_in, W0, b0, W1, b1, W2, b2, Wd, bd, Wc, bc):
    n = x.shape[0]
    e = edge_index.shape[1]
    src = edge_index[0].astype(jnp.int32)
    dst = edge_index[1].astype(jnp.int32)

    per_tile = -(-e // (NS * 4 * CHUNK)) * 4 * CHUNK
    e_pad = per_tile * NS
    iters = per_tile // CHUNK
    er = e_pad // 128
    padlen = e_pad - e
    pad0 = jnp.zeros((padlen,), jnp.int32)
    padn = jnp.full((padlen,), n, jnp.int32)
    src_g = jnp.concatenate([src, pad0])
    src_d = jnp.concatenate([src, padn])
    dst_p = jnp.concatenate([dst, padn])
    src2 = jnp.stack([src_g, src_g + n]).reshape(2, er, 128)
    degidx = jnp.stack([src_d, dst_p]).reshape(2, er, 128)
    dst2 = dst_p.reshape(er, 128)

    nacc = 8 * NS * (-(-(n + 1) // (8 * NS)))
    zr = nacc // NS
    zeros16 = jnp.zeros((zr, 16), jnp.float32)
    zeros32 = jnp.zeros((zr, 32), jnp.float32)
    ones16 = jnp.ones((128, 16), jnp.float32)

    deg = _sc_degree(degidx, ones16, zeros16, n=n, nacc=nacc, iters=iters)
    degp = jnp.broadcast_to(deg[:, :, 0:1], (2, n, 32)).reshape(2, n // 4, 128)
    degsp = degp[0]
    degdp = degp[1]

    eye4 = jnp.eye(4, dtype=jnp.float32)

    def bd4(a):
        return jnp.kron(eye4, a)

    def bd_w64(w):
        wq = w.reshape(2, 32, 2, 32)
        return jnp.stack([jnp.stack([bd4(wq[h, :, c, :]) for c in (0, 1)])
                          for h in (0, 1)])

    def b128(b):
        return jnp.tile(b.reshape(2, 1, 32), (1, 1, 4))

    x4 = x.reshape(n // 4, 4 * x.shape[1])
    winq = W_in.reshape(-1, 2, 32)
    winbd = jnp.stack([bd4(winq[:, c, :]) for c in (0, 1)])
    w0bd, w1bd, w2bd, wdbd = (bd_w64(w) for w in (W0, W1, W2, Wd))
    wcq = Wc.reshape(2, 32, -1)
    wcbd = jnp.stack([bd4(wcq[c]) for c in (0, 1)])
    bcx = jnp.tile(bc, 4).reshape(1, -1)

    h = _tc_pre(x4, winbd, b128(b_in), degsp, n)
    for wbd_, b_ in ((w0bd, b0), (w1bd, b1)):
        a = _sc_gather_scatter(h.reshape(2 * n, 32), src2, dst2, zeros32,
                               n=n, nacc=nacc, iters=iters)
        h = _tc_layer(a.reshape(2, n // 4, 128), wbd_, b128(b_),
                      degdp, degsp, n)
    a = _sc_gather_scatter(h.reshape(2 * n, 32), src2, dst2, zeros32,
                           n=n, nacc=nacc, iters=iters)
    y4 = _tc_final(a.reshape(2, n // 4, 128), w2bd, b128(b2), wdbd, b128(bd),
                   wcbd, bcx, degdp, n)
    return y4.reshape(n, -1)


# speedup vs baseline: 1.0696x; 1.0696x over previous
import functools

import jax
import jax.numpy as jnp
from jax import lax
from jax.experimental import pallas as pl
from jax.experimental.pallas import tpu as pltpu
from jax.experimental.pallas import tpu_sc as plsc

NC = 2
NS = 16
SUB = 3
CHUNK = SUB * 128


def _sc_mesh():
    return plsc.VectorSubcoreMesh(core_axis_name="c", subcore_axis_name="s")


_SC_PARAMS = pltpu.CompilerParams(use_tc_tiling_on_sc=False)


@functools.partial(jax.jit, static_argnames=("n", "nacc", "iters"))
def _sc_degree(idx2, ones_h, zeros_h, *, n, nacc, iters):
    zr = nacc // NS
    rows_per_tile = iters * SUB

    def body(idx_hbm, ones_hbm, zeros_hbm, out_hbm, idxv, ones_v, acc, sem):
        c = lax.axis_index("c")
        s = lax.axis_index("s")
        pltpu.sync_copy(ones_hbm, ones_v)
        pltpu.sync_copy(zeros_hbm, acc.at[pl.ds(s * zr, zr)])
        plsc.subcore_barrier()

        def idx_src(i):
            return idx_hbm.at[c, pl.ds(s * rows_per_tile + i * SUB, SUB)]

        pltpu.sync_copy(idx_src(0), idxv.at[0])

        def pair(p, carry):
            for b in (0, 1):
                i = 2 * p + b

                @pl.when(i + 1 < iters)
                def _():
                    pltpu.async_copy(idx_src(i + 1), idxv.at[1 - b], sem)

                for j in range(SUB):
                    pltpu.sync_copy(ones_v, acc.at[idxv.at[b, j]], add=True)

                @pl.when(i + 1 < iters)
                def _():
                    pltpu.make_async_copy(idx_src(i + 1), idxv.at[1 - b],
                                          sem).wait()
            return carry

        lax.fori_loop(0, iters // 2, pair, 0)
        plsc.subcore_barrier()
        r0 = s * zr
        pltpu.sync_copy(acc.at[pl.ds(r0, zr)], out_hbm.at[c, pl.ds(r0, zr)])

    fn = pl.kernel(
        body,
        out_type=jax.ShapeDtypeStruct((2, n, 16), jnp.float32),
        mesh=_sc_mesh(),
        scratch_types=[
            pltpu.VMEM((2, SUB, 128), jnp.int32),
            pltpu.VMEM((128, 16), jnp.float32),
            pltpu.VMEM_SHARED((nacc, 16), jnp.float32),
            pltpu.SemaphoreType.DMA,
        ],
        compiler_params=_SC_PARAMS,
    )
    return fn(idx2, ones_h, zeros_h)


@functools.partial(jax.jit, static_argnames=("n", "nacc", "iters"))
def _sc_gather_scatter(h2, src2, dst2, zeros_h, *, n, nacc, iters):
    zr = nacc // NS
    rows_per_tile = iters * SUB

    def body(h_hbm, src_hbm, dst_hbm, zeros_hbm, out_hbm,
             sidx, didx, rows_v, acc, gsem):
        c = lax.axis_index("c")
        s = lax.axis_index("s")
        pltpu.sync_copy(zeros_hbm, acc.at[pl.ds(s * zr, zr)])
        plsc.subcore_barrier()

        def load_idx(i, q, b):
            base = s * rows_per_tile + i * SUB
            pltpu.sync_copy(src_hbm.at[c, pl.ds(base, SUB)], sidx.at[q, b])
            pltpu.sync_copy(dst_hbm.at[pl.ds(base, SUB)], didx.at[q, b])

        def fire_gathers(q, b):
            for j in range(SUB):
                pltpu.async_copy(h_hbm.at[sidx.at[q, b, j]],
                                 rows_v.at[b, pl.ds(j * 128, 128)], gsem)

        def drain_gathers(b):
            pltpu.make_async_copy(h_hbm.at[pl.ds(0, CHUNK)], rows_v.at[b],
                                  gsem).wait()

        load_idx(0, 0, 0)
        load_idx(1, 0, 1)
        fire_gathers(0, 0)

        def quad(p2, carry):
            for pp in (0, 1):
                for b in (0, 1):
                    i = 4 * p2 + 2 * pp + b
                    q = pp
                    nq, nb = (q, 1 - b) if b == 0 else (1 - q, 1 - b)

                    @pl.when(i + 1 < iters)
                    def _():
                        fire_gathers(nq, nb)

                    @pl.when(i + 2 < iters)
                    def _():
                        load_idx(i + 2, 1 - q, b)

                    drain_gathers(b)
                    for j in range(SUB):
                        pltpu.sync_copy(rows_v.at[b, pl.ds(j * 128, 128)],
                                        acc.at[didx.at[q, b, j]], add=True)
            return carry

        lax.fori_loop(0, iters // 4, quad, 0)
        plsc.subcore_barrier()
        r0 = s * zr
        pltpu.sync_copy(acc.at[pl.ds(r0, zr)], out_hbm.at[c, pl.ds(r0, zr)])

    fn = pl.kernel(
        body,
        out_type=jax.ShapeDtypeStruct((2, n, 32), jnp.float32),
        mesh=_sc_mesh(),
        scratch_types=[
            pltpu.VMEM((2, 2, SUB, 128), jnp.int32),
            pltpu.VMEM((2, 2, SUB, 128), jnp.int32),
            pltpu.VMEM((2, CHUNK, 32), jnp.float32),
            pltpu.VMEM_SHARED((nacc, 32), jnp.float32),
            pltpu.SemaphoreType.DMA,
        ],
        compiler_params=_SC_PARAMS,
    )
    return fn(h2, src2, dst2, zeros_h)


_RP = 256


def _normp(deg_blk):
    return lax.rsqrt(jnp.maximum(deg_blk, 1.0))


def _tc_pre_body(x_ref, w_ref, b_ref, d_ref, o_ref):
    xb = x_ref[...]
    ns = _normp(d_ref[...])
    for c in (0, 1):
        h = jnp.tanh(jnp.dot(xb, w_ref[c], preferred_element_type=jnp.float32)
                     + b_ref[c])
        o_ref[c] = h * ns


def _tc_layer_body(a_ref, w_ref, b_ref, dd_ref, ds_ref, o_ref):
    nd = _normp(dd_ref[...])
    ns = _normp(ds_ref[...])
    a0 = a_ref[0] * nd
    a1 = a_ref[1] * nd
    for c in (0, 1):
        h = jnp.tanh(jnp.dot(a0, w_ref[0, c], preferred_element_type=jnp.float32)
                     + jnp.dot(a1, w_ref[1, c], preferred_element_type=jnp.float32)
                     + b_ref[c])
        o_ref[c] = h * ns


def _tc_final_body(a_ref, w2_ref, b2_ref, wd_ref, bd_ref, wc_ref, bc_ref,
                   dd_ref, o_ref):
    nd = _normp(dd_ref[...])
    a0 = a_ref[0] * nd
    a1 = a_ref[1] * nd
    g = [jnp.dot(a0, w2_ref[0, c], preferred_element_type=jnp.float32)
         + jnp.dot(a1, w2_ref[1, c], preferred_element_type=jnp.float32)
         + b2_ref[c] for c in (0, 1)]
    t = [jnp.tanh(jnp.dot(g[0], wd_ref[0, c], preferred_element_type=jnp.float32)
                  + jnp.dot(g[1], wd_ref[1, c], preferred_element_type=jnp.float32)
                  + bd_ref[c]) for c in (0, 1)]
    o_ref[...] = (jnp.dot(t[0], wc_ref[0], preferred_element_type=jnp.float32)
                  + jnp.dot(t[1], wc_ref[1], preferred_element_type=jnp.float32)
                  + bc_ref[...])


def _rp_spec(cols):
    return pl.BlockSpec((_RP, cols), lambda i: (i, 0))


def _fullb(*shape):
    nd = len(shape)
    return pl.BlockSpec(shape, lambda i, _nd=nd: (0,) * _nd)


def _packed_spec():
    return pl.BlockSpec((2, _RP, 128), lambda i: (0, i, 0))


def _tc_pre(x4, wbd, b2x, degp, n):
    return pl.pallas_call(
        _tc_pre_body,
        grid=(-(-(n // 4) // _RP),),
        in_specs=[_rp_spec(x4.shape[1]), _fullb(*wbd.shape),
                  _fullb(2, 1, 128), _rp_spec(128)],
        out_specs=_packed_spec(),
        out_shape=jax.ShapeDtypeStruct((2, n // 4, 128), jnp.float32),
    )(x4, wbd, b2x, degp)


def _tc_layer(a, wbd, b2x, degdp, degsp, n):
    return pl.pallas_call(
        _tc_layer_body,
        grid=(-(-(n // 4) // _RP),),
        in_specs=[_packed_spec(), _fullb(*wbd.shape), _fullb(2, 1, 128),
                  _rp_spec(128), _rp_spec(128)],
        out_specs=_packed_spec(),
        out_shape=jax.ShapeDtypeStruct((2, n // 4, 128), jnp.float32),
    )(a, wbd, b2x, degdp, degsp)


def _tc_final(a, w2bd, b2x, wdbd, bdx, wcbd, bcx, degdp, n):
    cols = wcbd.shape[2]
    return pl.pallas_call(
        _tc_final_body,
        grid=(-(-(n // 4) // _RP),),
        in_specs=[_packed_spec(), _fullb(*w2bd.shape), _fullb(2, 1, 128),
                  _fullb(*wdbd.shape), _fullb(2, 1, 128),
                  _fullb(*wcbd.shape), _fullb(1, cols), _rp_spec(128)],
        out_specs=pl.BlockSpec((_RP, cols), lambda i: (i, 0)),
        out_shape=jax.ShapeDtypeStruct((n // 4, cols), jnp.float32),
    )(a, w2bd, b2x, wdbd, bdx, wcbd, bcx, degdp)


def kernel(x, edge_index, W_in, b_in, W0, b0, W1, b1, W2, b2, Wd, bd, Wc, bc):
    n = x.shape[0]
    e = edge_index.shape[1]
    src = edge_index[0].astype(jnp.int32)
    dst = edge_index[1].astype(jnp.int32)

    n2 = 128 * (-(-n // 128))

    per_tile = -(-e // (NS * 4 * CHUNK)) * 4 * CHUNK
    e_pad = per_tile * NS
    iters = per_tile // CHUNK
    er = e_pad // 128
    padlen = e_pad - e
    pad0 = jnp.zeros((padlen,), jnp.int32)
    padn = jnp.full((padlen,), n, jnp.int32)
    src_g = jnp.concatenate([src, pad0])
    src_d = jnp.concatenate([src, padn])
    dst_p = jnp.concatenate([dst, padn])
    src2 = jnp.stack([src_g, src_g + n2]).reshape(2, er, 128)
    degidx = jnp.stack([src_d, dst_p]).reshape(2, er, 128)
    dst2 = dst_p.reshape(er, 128)

    nacc = n2
    zr = nacc // NS
    zeros16 = jnp.zeros((zr, 16), jnp.float32)
    zeros32 = jnp.zeros((zr, 32), jnp.float32)
    ones16 = jnp.ones((128, 16), jnp.float32)

    deg = _sc_degree(degidx, ones16, zeros16, n=n2, nacc=nacc, iters=iters)
    degp = jnp.broadcast_to(deg[:, :, 0:1], (2, n2, 32)).reshape(2, n2 // 4, 128)
    degsp = degp[0]
    degdp = degp[1]

    eye4 = jnp.eye(4, dtype=jnp.float32)

    def bd4(a):
        return jnp.kron(eye4, a)

    def bd_w64(w):
        wq = w.reshape(2, 32, 2, 32)
        return jnp.stack([jnp.stack([bd4(wq[h, :, c, :]) for c in (0, 1)])
                          for h in (0, 1)])

    def b128(b):
        return jnp.tile(b.reshape(2, 1, 32), (1, 1, 4))

    x4 = x.reshape(n // 4, 4 * x.shape[1])
    winq = W_in.reshape(-1, 2, 32)
    winbd = jnp.stack([bd4(winq[:, c, :]) for c in (0, 1)])
    w0bd, w1bd, w2bd, wdbd = (bd_w64(w) for w in (W0, W1, W2, Wd))
    wcq = Wc.reshape(2, 32, -1)
    wcbd = jnp.stack([bd4(wcq[c]) for c in (0, 1)])
    bcx = jnp.tile(bc, 4).reshape(1, -1)

    h = _tc_pre(x4, winbd, b128(b_in), degsp, n2)
    for wbd_, b_ in ((w0bd, b0), (w1bd, b1)):
        a = _sc_gather_scatter(h.reshape(2 * n2, 32), src2, dst2, zeros32,
                               n=n2, nacc=nacc, iters=iters)
        h = _tc_layer(a.reshape(2, n2 // 4, 128), wbd_, b128(b_),
                      degdp, degsp, n2)
    a = _sc_gather_scatter(h.reshape(2 * n2, 32), src2, dst2, zeros32,
                           n=n2, nacc=nacc, iters=iters)
    y4 = _tc_final(a.reshape(2, n2 // 4, 128), w2bd, b128(b2), wdbd, b128(bd),
                   wcbd, bcx, degdp, n2)
    return y4[:n // 4].reshape(n, -1)
